# Initial kernel scaffold; baseline (speedup 1.0000x reference)
import jax
import jax.numpy as jnp
from jax.experimental import pallas as pl


def kernel(hidden_states, layer_idx, gate_w, w1, w3, w2):
    raise NotImplementedError("write your pallas kernel here")


# speedup vs baseline: 2.6755x; 2.6755x over previous
import jax
import jax.numpy as jnp
from jax.experimental import pallas as pl

E = 64
TOP_K = 8
D_MODEL = 1024
D_FF = 512
N_GROUP = 8
TOPK_GROUP = 4
T = 2048
GS = E // N_GROUP


def _routing_kernel(x_ref, gw_ref, w_ref):
    x = x_ref[...]
    gw = gw_ref[...]
    logits = jax.lax.dot_general(x, gw, (((1,), (1,)), ((), ())),
                                 preferred_element_type=jnp.float32)
    m = jnp.max(logits, axis=1, keepdims=True)
    ex = jnp.exp(logits - m)
    scores = ex / jnp.sum(ex, axis=1, keepdims=True)

    lane = jax.lax.broadcasted_iota(jnp.int32, (T, E), 1)
    group_of_lane = lane // GS

    G = jnp.zeros((T, E), jnp.float32)
    gmaxes = []
    for g in range(N_GROUP):
        gm = jnp.max(jnp.where(group_of_lane == g, scores, -jnp.inf), axis=1,
                     keepdims=True)
        gmaxes.append(gm)
        G = jnp.where(group_of_lane == g, gm, G)

    rank = jnp.zeros((T, E), jnp.int32)
    for g in range(N_GROUP):
        gm = gmaxes[g]
        rank = rank + jnp.where(gm > G, 1, 0) \
                    + jnp.where((gm == G) & (g < group_of_lane), 1, 0)
    ms = jnp.where(rank < TOPK_GROUP, scores, 0.0)

    chosen = jnp.zeros((T, E), jnp.bool_)
    work = ms
    denom = jnp.zeros((T, 1), jnp.float32)
    for _ in range(TOP_K):
        mx = jnp.max(work, axis=1, keepdims=True)
        pick_lane = jnp.min(jnp.where(work == mx, lane, E), axis=1,
                            keepdims=True)
        pick = lane == pick_lane
        chosen = chosen | pick
        denom = denom + mx
        work = jnp.where(pick, -1.0, work)

    w_ref[...] = jnp.where(chosen, ms, 0.0) / (denom + 1e-20)


def _moe_kernel(x_ref, w_ref, w1_ref, w3_ref, w2_ref, out_ref):
    e = pl.program_id(0)
    x = x_ref[...]
    lane = jax.lax.broadcasted_iota(jnp.int32, (T, E), 1)
    wcol = jnp.sum(jnp.where(lane == e, w_ref[...], 0.0), axis=1,
                   keepdims=True)
    w1e = w1_ref[0]
    w3e = w3_ref[0]
    w2e = w2_ref[0]
    h1 = jax.lax.dot_general(x, w1e, (((1,), (1,)), ((), ())),
                             preferred_element_type=jnp.float32)
    h3 = jax.lax.dot_general(x, w3e, (((1,), (1,)), ((), ())),
                             preferred_element_type=jnp.float32)
    h = (h1 * jax.nn.sigmoid(h1)) * h3
    y = jax.lax.dot_general(h, w2e, (((1,), (1,)), ((), ())),
                            preferred_element_type=jnp.float32)

    @pl.when(e == 0)
    def _():
        out_ref[...] = jnp.zeros_like(out_ref)

    out_ref[...] += wcol * y


def kernel(hidden_states, layer_idx, gate_w, w1, w3, w2):
    del layer_idx
    combine_w = pl.pallas_call(
        _routing_kernel,
        out_shape=jax.ShapeDtypeStruct((T, E), jnp.float32),
    )(hidden_states, gate_w)

    out = pl.pallas_call(
        _moe_kernel,
        grid=(E,),
        in_specs=[
            pl.BlockSpec((T, D_MODEL), lambda e: (0, 0)),
            pl.BlockSpec((T, E), lambda e: (0, 0)),
            pl.BlockSpec((1, D_FF, D_MODEL), lambda e: (e, 0, 0)),
            pl.BlockSpec((1, D_FF, D_MODEL), lambda e: (e, 0, 0)),
            pl.BlockSpec((1, D_MODEL, D_FF), lambda e: (e, 0, 0)),
        ],
        out_specs=pl.BlockSpec((T, D_MODEL), lambda e: (0, 0)),
        out_shape=jax.ShapeDtypeStruct((T, D_MODEL), jnp.float32),
    )(hidden_states, combine_w, w1, w3, w2)
    return out
